# baseline (speedup 1.0000x reference)
import functools

import jax
import jax.numpy as jnp
from jax import lax
from jax.experimental import pallas as pl
from jax.experimental.pallas import tpu as pltpu
from jax.experimental.pallas import tpu_sc as plsc

N = 16384
Q = 1000
QPAD = 1024
NTILES = 16
ROWS_P1 = N // NTILES
NW = 32
ROWS_P2 = N // NW
ZCH = QPAD // NTILES


def _sc_body(idx3, mm0c, mm1c, mm2c, xf, betas, out_hbm,
             idxv, val0, val1, val2, onesv, zerov,
             tab0, tab1, tab2, tab3, pidxv, xv, bv, outv,
             tsh0, tsh1, tsh2, tsh3):
    vals = (val0, val1, val2)
    tabs = (tab0, tab1, tab2, tab3)
    tshs = (tsh0, tsh1, tsh2, tsh3)
    c = lax.axis_index("c")
    s = lax.axis_index("s")
    base = s * ROWS_P1 + c * ROWS_P2

    zero16 = jnp.zeros((16,), jnp.float32)
    one16 = jnp.ones((16,), jnp.float32)

    def fill_z(i, _):
        zerov[pl.ds(i * 16, 16)] = zero16
        return 0
    lax.fori_loop(0, ZCH // 16, fill_z, 0)

    def fill_o(i, _):
        onesv[pl.ds(i * 16, 16)] = one16
        return 0
    lax.fori_loop(0, 128 // 16, fill_o, 0)

    for t in range(4):
        pltpu.sync_copy(zerov, tshs[t].at[pl.ds(s * ZCH, ZCH)])
    plsc.subcore_barrier()

    pltpu.sync_copy(idx3.at[s], idxv)
    for t, mmc in enumerate((mm0c, mm1c, mm2c)):
        pltpu.sync_copy(mmc.at[pl.ds(s * ROWS_P1, ROWS_P1)],
                        vals[t])
    for j in range(8):
        for t in range(3):
            pltpu.sync_copy(vals[t].at[pl.ds(j * 128, 128)],
                            tshs[t].at[idxv.at[j]], add=True)
        pltpu.sync_copy(onesv, tshs[3].at[idxv.at[j]], add=True)
    plsc.subcore_barrier()

    for t in range(4):
        pltpu.sync_copy(tshs[t], tabs[t])
    pltpu.sync_copy(idx3.at[s, pl.ds(c * 4, 4)], pidxv)
    pltpu.sync_copy(xf.at[pl.ds(base, ROWS_P2)], xv)
    pltpu.sync_copy(betas, bv)
    b1 = bv[0]
    b2 = bv[1]
    b3 = bv[2]

    def p2(k, _):
        g = k // 8
        l = k % 8
        iv = pidxv[g, pl.ds(l * 16, 16)]
        s0 = plsc.load_gather(tab0, [iv])
        s1 = plsc.load_gather(tab1, [iv])
        s2 = plsc.load_gather(tab2, [iv])
        cn = plsc.load_gather(tab3, [iv])
        empty = cn == 0.0
        safe = jnp.where(empty, 1.0, cn)
        z0 = jnp.where(empty, 0.0, s0 / safe)
        z1 = jnp.where(empty, 0.0, s1 / safe)
        z2 = jnp.where(empty, 0.0, s2 / safe)
        x = xv[pl.ds(k * 16, 16)]
        t = (x - (b2 + z1)) / jnp.maximum(b3 + z2, jnp.float32(0.1))
        e = jnp.exp(jnp.clip(-t, -50.0, 50.0))
        outv[pl.ds(k * 16, 16)] = (b1 + z0) / (1.0 + e)
    for k in range(ROWS_P2 // 16):
        p2(k, None)
    pltpu.sync_copy(outv, out_hbm.at[pl.ds(base, ROWS_P2)])


_sc_call = functools.partial(
    pl.kernel,
    out_type=jax.ShapeDtypeStruct((N,), jnp.float32),
    mesh=plsc.VectorSubcoreMesh(core_axis_name="c", subcore_axis_name="s"),
    compiler_params=pltpu.CompilerParams(needs_layout_passes=False),
    scratch_types=[
        pltpu.VMEM((8, 128), jnp.int32),
        pltpu.VMEM((ROWS_P1,), jnp.float32),
        pltpu.VMEM((ROWS_P1,), jnp.float32),
        pltpu.VMEM((ROWS_P1,), jnp.float32),
        pltpu.VMEM((128,), jnp.float32),
        pltpu.VMEM((ZCH,), jnp.float32),
        pltpu.VMEM((QPAD,), jnp.float32),
        pltpu.VMEM((QPAD,), jnp.float32),
        pltpu.VMEM((QPAD,), jnp.float32),
        pltpu.VMEM((QPAD,), jnp.float32),
        pltpu.VMEM((4, 128), jnp.int32),
        pltpu.VMEM((ROWS_P2,), jnp.float32),
        pltpu.VMEM((3, 16), jnp.float32),
        pltpu.VMEM((ROWS_P2,), jnp.float32),
        pltpu.VMEM_SHARED((QPAD,), jnp.float32),
        pltpu.VMEM_SHARED((QPAD,), jnp.float32),
        pltpu.VMEM_SHARED((QPAD,), jnp.float32),
        pltpu.VMEM_SHARED((QPAD,), jnp.float32),
    ],
)(_sc_body)


_BN = 2048


def _onehot_body(i0_ref, i1_ref, o0_ref, o1_ref):
    iota = lax.broadcasted_iota(jnp.int32, (_BN, 1024), 1)
    o0_ref[...] = (i0_ref[...][:, None] == iota).astype(jnp.float32)
    o1_ref[...] = (i1_ref[...][:, None] == iota).astype(jnp.float32)


_onehot_call = pl.pallas_call(
    _onehot_body,
    grid=(N // _BN,),
    in_specs=[pl.BlockSpec((_BN,), lambda i: (i,)),
              pl.BlockSpec((_BN,), lambda i: (i,))],
    out_specs=[pl.BlockSpec((_BN, 1024), lambda i: (i, 0)),
               pl.BlockSpec((_BN, 1024), lambda i: (i, 0))],
    out_shape=[jax.ShapeDtypeStruct((N, 1024), jnp.float32),
               jax.ShapeDtypeStruct((N, 1024), jnp.float32)],
)


def kernel(X_input, Z_inputs_0, Z_inputs_1, mmbeddings_0, mmbeddings_1,
           beta_1, beta_2, beta_3):
    del mmbeddings_1
    i0 = Z_inputs_0.astype(jnp.int32)
    i1 = Z_inputs_1.astype(jnp.int32)
    idx3 = i0.reshape(NTILES, 8, 128)
    mm0c = mmbeddings_0[:, 0]
    mm1c = mmbeddings_0[:, 1]
    mm2c = mmbeddings_0[:, 2]
    betas = jnp.stack([
        jnp.full((16,), beta_1, jnp.float32),
        jnp.full((16,), beta_2, jnp.float32),
        jnp.full((16,), beta_3, jnp.float32),
    ])
    out_flat = _sc_call(idx3, mm0c, mm1c, mm2c, X_input.reshape(N), betas)
    oh0p, oh1p = _onehot_call(i0, i1)
    return out_flat.reshape(N, 1), oh0p[:, :Q], oh1p[:, :Q]


# speedup vs baseline: 1.0126x; 1.0126x over previous
import functools

import jax
import jax.numpy as jnp
from jax import lax
from jax.experimental import pallas as pl
from jax.experimental.pallas import tpu as pltpu
from jax.experimental.pallas import tpu_sc as plsc

N = 16384
Q = 1000
QPAD = 1024
NTILES = 16
ROWS_P1 = N // NTILES
NW = 32
ROWS_P2 = N // NW
ZCH = QPAD // NTILES


def _sc_body(idx3, mm0c, mm1c, mm2c, xf, betas, out_hbm,
             idxv, val0, val1, val2, onesv, zerov,
             tab0, tab1, tab2, tab3, pidxv, xv, bv, outv,
             tsh0, tsh1, tsh2, tsh3, sem_p1):
    vals = (val0, val1, val2)
    tabs = (tab0, tab1, tab2, tab3)
    tshs = (tsh0, tsh1, tsh2, tsh3)
    c = lax.axis_index("c")
    s = lax.axis_index("s")
    base = s * ROWS_P1 + c * ROWS_P2

    zero16 = jnp.zeros((16,), jnp.float32)
    one16 = jnp.ones((16,), jnp.float32)

    def fill_z(i, _):
        zerov[pl.ds(i * 16, 16)] = zero16
        return 0
    lax.fori_loop(0, ZCH // 16, fill_z, 0)

    def fill_o(i, _):
        onesv[pl.ds(i * 16, 16)] = one16
        return 0
    lax.fori_loop(0, 128 // 16, fill_o, 0)

    for t in range(4):
        pltpu.sync_copy(zerov, tshs[t].at[pl.ds(s * ZCH, ZCH)])
    plsc.subcore_barrier()

    pltpu.sync_copy(idx3.at[s], idxv)
    for t, mmc in enumerate((mm0c, mm1c, mm2c)):
        pltpu.sync_copy(mmc.at[pl.ds(s * ROWS_P1, ROWS_P1)],
                        vals[t])
    descs = []
    for j in range(8):
        for t in range(3):
            descs.append(pltpu.async_copy(
                vals[t].at[pl.ds(j * 128, 128)],
                tshs[t].at[idxv.at[j]], sem_p1, add=True))
        descs.append(pltpu.async_copy(
            onesv, tshs[3].at[idxv.at[j]], sem_p1, add=True))
    for d in descs:
        d.wait()
    plsc.subcore_barrier()

    for t in range(4):
        pltpu.sync_copy(tshs[t], tabs[t])
    pltpu.sync_copy(idx3.at[s, pl.ds(c * 4, 4)], pidxv)
    pltpu.sync_copy(xf.at[pl.ds(base, ROWS_P2)], xv)
    pltpu.sync_copy(betas, bv)
    b1 = bv[0]
    b2 = bv[1]
    b3 = bv[2]

    def p2(k, _):
        g = k // 8
        l = k % 8
        iv = pidxv[g, pl.ds(l * 16, 16)]
        s0 = plsc.load_gather(tab0, [iv])
        s1 = plsc.load_gather(tab1, [iv])
        s2 = plsc.load_gather(tab2, [iv])
        cn = plsc.load_gather(tab3, [iv])
        empty = cn == 0.0
        safe = jnp.where(empty, 1.0, cn)
        z0 = jnp.where(empty, 0.0, s0 / safe)
        z1 = jnp.where(empty, 0.0, s1 / safe)
        z2 = jnp.where(empty, 0.0, s2 / safe)
        x = xv[pl.ds(k * 16, 16)]
        t = (x - (b2 + z1)) / jnp.maximum(b3 + z2, jnp.float32(0.1))
        e = jnp.exp(jnp.clip(-t, -50.0, 50.0))
        outv[pl.ds(k * 16, 16)] = (b1 + z0) / (1.0 + e)
    for k in range(ROWS_P2 // 16):
        p2(k, None)
    pltpu.sync_copy(outv, out_hbm.at[pl.ds(base, ROWS_P2)])


_sc_call = functools.partial(
    pl.kernel,
    out_type=jax.ShapeDtypeStruct((N,), jnp.float32),
    mesh=plsc.VectorSubcoreMesh(core_axis_name="c", subcore_axis_name="s"),
    compiler_params=pltpu.CompilerParams(needs_layout_passes=False),
    scratch_types=[
        pltpu.VMEM((8, 128), jnp.int32),
        pltpu.VMEM((ROWS_P1,), jnp.float32),
        pltpu.VMEM((ROWS_P1,), jnp.float32),
        pltpu.VMEM((ROWS_P1,), jnp.float32),
        pltpu.VMEM((128,), jnp.float32),
        pltpu.VMEM((ZCH,), jnp.float32),
        pltpu.VMEM((QPAD,), jnp.float32),
        pltpu.VMEM((QPAD,), jnp.float32),
        pltpu.VMEM((QPAD,), jnp.float32),
        pltpu.VMEM((QPAD,), jnp.float32),
        pltpu.VMEM((4, 128), jnp.int32),
        pltpu.VMEM((ROWS_P2,), jnp.float32),
        pltpu.VMEM((3, 16), jnp.float32),
        pltpu.VMEM((ROWS_P2,), jnp.float32),
        pltpu.VMEM_SHARED((QPAD,), jnp.float32),
        pltpu.VMEM_SHARED((QPAD,), jnp.float32),
        pltpu.VMEM_SHARED((QPAD,), jnp.float32),
        pltpu.VMEM_SHARED((QPAD,), jnp.float32),
        pltpu.SemaphoreType.DMA,
    ],
)(_sc_body)


_BN = 2048


def _onehot_body(i0_ref, i1_ref, o0_ref, o1_ref):
    iota = lax.broadcasted_iota(jnp.int32, (_BN, 1024), 1)
    o0_ref[...] = (i0_ref[...][:, None] == iota).astype(jnp.float32)
    o1_ref[...] = (i1_ref[...][:, None] == iota).astype(jnp.float32)


_onehot_call = pl.pallas_call(
    _onehot_body,
    grid=(N // _BN,),
    in_specs=[pl.BlockSpec((_BN,), lambda i: (i,)),
              pl.BlockSpec((_BN,), lambda i: (i,))],
    out_specs=[pl.BlockSpec((_BN, 1024), lambda i: (i, 0)),
               pl.BlockSpec((_BN, 1024), lambda i: (i, 0))],
    out_shape=[jax.ShapeDtypeStruct((N, 1024), jnp.float32),
               jax.ShapeDtypeStruct((N, 1024), jnp.float32)],
)


def kernel(X_input, Z_inputs_0, Z_inputs_1, mmbeddings_0, mmbeddings_1,
           beta_1, beta_2, beta_3):
    del mmbeddings_1
    i0 = Z_inputs_0.astype(jnp.int32)
    i1 = Z_inputs_1.astype(jnp.int32)
    idx3 = i0.reshape(NTILES, 8, 128)
    mm0c = mmbeddings_0[:, 0]
    mm1c = mmbeddings_0[:, 1]
    mm2c = mmbeddings_0[:, 2]
    betas = jnp.stack([
        jnp.full((16,), beta_1, jnp.float32),
        jnp.full((16,), beta_2, jnp.float32),
        jnp.full((16,), beta_3, jnp.float32),
    ])
    out_flat = _sc_call(idx3, mm0c, mm1c, mm2c, X_input.reshape(N), betas)
    oh0p, oh1p = _onehot_call(i0, i1)
    return out_flat.reshape(N, 1), oh0p[:, :Q], oh1p[:, :Q]
